# baseline (speedup 1.0000x reference)
import jax
import jax.numpy as jnp
from jax import lax
from jax.experimental import pallas as pl
from jax.experimental.pallas import tpu as pltpu
from jax.experimental.pallas import tpu_sc as plsc

N = 10000
F = 256
H = 256
HH = 128
N_PAD = 10240
E_PAD = 163840
CH = 128
NCH = 80
NB = 8
LEAD = 4
EPT = NCH * CH
RPT = N_PAD // 16


def _z16():
  return jnp.zeros((16,), jnp.float32)


def _sc_agg_body(xcat, ed3, agg_out, deg_out,
                 ed_v, rows_v, misc_v, agg_sp, deg_sp, sem, sem_s):
  cid = lax.axis_index("c")
  sid = lax.axis_index("s")

  pltpu.sync_copy(ed3.at[0, sid], ed_v.at[0])
  pltpu.sync_copy(ed3.at[1, sid], ed_v.at[1])

  def _zrow(j, _):
    def _zlane(l, _):
      rows_v[0, j, pl.ds(l * 32, 32)] = jnp.zeros((32,), jnp.bfloat16)
      return 0
    return lax.fori_loop(0, HH // 32, _zlane, 0)
  lax.fori_loop(0, CH, _zrow, 0)

  def _zv(k, _):
    misc_v[pl.ds(k * 16, 16)] = _z16()
    return 0
  lax.fori_loop(0, RPT // 16, _zv, 0)

  def _ov(k, _):
    misc_v[pl.ds(RPT + k * 16, 16)] = jnp.ones((16,), jnp.float32)
    return 0
  lax.fori_loop(0, CH // 16, _ov, 0)

  for r in range(RPT // CH):
    pltpu.sync_copy(rows_v.at[0], agg_sp.at[pl.ds(sid * RPT + r * CH, CH)])
  pltpu.sync_copy(misc_v.at[pl.ds(0, RPT)], deg_sp.at[pl.ds(sid * RPT, RPT)])

  off = (cid * N).astype(jnp.int32)
  def _adjj(j, _):
    def _adjl(l, _):
      ed_v[0, j, pl.ds(l * 16, 16)] = ed_v[0, j, pl.ds(l * 16, 16)] + off
      return 0
    return lax.fori_loop(0, CH // 16, _adjl, 0)
  lax.fori_loop(0, NCH, _adjj, 0)

  plsc.subcore_barrier()

  for b in range(LEAD):
    pltpu.async_copy(xcat.at[ed_v.at[0, b]], rows_v.at[b], sem)

  def _group(g, _):
    for b in range(NB):
      c = g * NB + b
      cl = c + LEAD
      bl = (b + LEAD) % NB

      @pl.when(cl < NCH)
      def _():
        @pl.when(c >= NB - LEAD)
        def _():
          pltpu.make_async_copy(rows_v.at[0], agg_sp.at[ed_v.at[1, 0]],
                                sem_s).wait()
        pltpu.async_copy(xcat.at[ed_v.at[0, cl]], rows_v.at[bl], sem)

      pltpu.make_async_copy(xcat.at[ed_v.at[0, c]], rows_v.at[b],
                            sem).wait()
      pltpu.async_copy(rows_v.at[b], agg_sp.at[ed_v.at[1, c]], sem_s,
                       add=True)
    return 0

  lax.fori_loop(0, NCH // NB, _group, 0)

  for _ in range(NB):
    pltpu.make_async_copy(rows_v.at[0], agg_sp.at[ed_v.at[1, 0]],
                          sem_s).wait()

  def _deg(j, _):
    pltpu.sync_copy(misc_v.at[pl.ds(RPT, CH)], deg_sp.at[ed_v.at[1, j]],
                    add=True)
    return 0
  lax.fori_loop(cid * (NCH // 2), (cid + 1) * (NCH // 2), _deg, 0)

  plsc.subcore_barrier()

  pltpu.sync_copy(agg_sp.at[pl.ds(sid * RPT, RPT)],
                  agg_out.at[cid, pl.ds(sid * RPT, RPT)])
  pltpu.sync_copy(deg_sp.at[pl.ds(sid * RPT, RPT)],
                  deg_out.at[cid, pl.ds(sid * RPT, RPT)])


_sc_agg = pl.kernel(
    _sc_agg_body,
    out_type=(
        jax.ShapeDtypeStruct((2, N_PAD, HH), jnp.bfloat16),
        jax.ShapeDtypeStruct((2, N_PAD), jnp.float32),
    ),
    mesh=plsc.VectorSubcoreMesh(core_axis_name="c", subcore_axis_name="s"),
    compiler_params=pltpu.CompilerParams(needs_layout_passes=False,
                                         use_tc_tiling_on_sc=False),
    scratch_types=[
        pltpu.VMEM((2, NCH, CH), jnp.int32),
        pltpu.VMEM((NB, CH, HH), jnp.bfloat16),
        pltpu.VMEM((RPT + CH,), jnp.float32),
        pltpu.VMEM_SHARED((N_PAD, HH), jnp.bfloat16),
        pltpu.VMEM_SHARED((N_PAD,), jnp.float32),
        pltpu.SemaphoreType.DMA,
        pltpu.SemaphoreType.DMA,
    ],
)


def _tc_mid_body(agg, deg, x, w1lt, w1rt, b1, w2lt, w2rt, b2,
                 s_out, u_out, invd_out):
  d = deg[0] + deg[1]
  invd = 1.0 / jnp.maximum(d, 1.0)
  m0 = agg[0].astype(jnp.float32) * invd
  m1 = agg[1].astype(jnp.float32) * invd
  a = jnp.dot(x[...], w1rt[...], preferred_element_type=jnp.float32)
  a = a + jnp.dot(m0, w1lt[0:HH, :], preferred_element_type=jnp.float32)
  a = a + jnp.dot(m1, w1lt[HH:F, :], preferred_element_type=jnp.float32)
  h = jnp.maximum(a + b1[...], 0.0)
  s_out[...] = jnp.dot(h, w2lt[...], preferred_element_type=jnp.float32)
  u_out[...] = jnp.dot(h, w2rt[...], preferred_element_type=jnp.float32) + b2[...]
  invd_out[...] = invd


_BM = 2000


def _tc_mid(agg, deg, x, w1lt, w1rt, b1, w2lt, w2rt, b2):
  grid = (N // _BM,)
  row = lambda i: (i, 0)
  rep = lambda i: (0, 0)
  return pl.pallas_call(
      _tc_mid_body,
      grid=grid,
      in_specs=[
          pl.BlockSpec((2, _BM, HH), lambda i: (0, i, 0)),
          pl.BlockSpec((2, _BM, 1), lambda i: (0, i, 0)),
          pl.BlockSpec((_BM, F), row),
          pl.BlockSpec((F, H), rep),
          pl.BlockSpec((F, H), rep),
          pl.BlockSpec((1, H), rep),
          pl.BlockSpec((H, 1), rep),
          pl.BlockSpec((H, 1), rep),
          pl.BlockSpec((1, 1), rep),
      ],
      out_specs=[
          pl.BlockSpec((_BM, 1), row),
          pl.BlockSpec((_BM, 1), row),
          pl.BlockSpec((_BM, 1), row),
      ],
      out_shape=[
          jax.ShapeDtypeStruct((N_PAD, 1), jnp.float32),
          jax.ShapeDtypeStruct((N_PAD, 1), jnp.float32),
          jax.ShapeDtypeStruct((N_PAD, 1), jnp.float32),
      ],
  )(agg, deg, x, w1lt, w1rt, b1, w2lt, w2rt, b2)


def _sc_out_body(s_pad, u_pad, invd_pad, ed2, out,
                 s_v, src_v, dst_v, hist_v, hsum_v, acc_v, stage_sp):
  cid = lax.axis_index("c")
  sid = lax.axis_index("s")

  pltpu.sync_copy(s_pad, s_v)
  pltpu.sync_copy(ed2.at[0, sid], src_v)
  pltpu.sync_copy(ed2.at[1, sid], dst_v)

  def _zhist(k, _):
    hist_v[pl.ds(k * 16, 16)] = _z16()
    return 0
  lax.fori_loop(0, N_PAD // 16, _zhist, 0)

  def _edge(k, _):
    idx = src_v[pl.ds(k * 16, 16)]
    d = dst_v[pl.ds(k * 16, 16)]
    v = plsc.load_gather(s_v, [idx])
    plsc.addupdate_scatter(hist_v, [d], v)
    return 0
  lax.fori_loop(0, EPT // 16, _edge, 0)

  pltpu.sync_copy(hist_v, stage_sp.at[sid])
  plsc.subcore_barrier()

  for r in range(16):
    pltpu.sync_copy(stage_sp.at[r, pl.ds(sid * RPT, RPT)], hsum_v.at[r])
  pltpu.sync_copy(invd_pad.at[pl.ds(sid * RPT, RPT)], acc_v.at[0])
  pltpu.sync_copy(u_pad.at[pl.ds(sid * RPT, RPT)], acc_v.at[1])

  def _fin(k, _):
    sl = pl.ds(k * 16, 16)
    v = hsum_v[0, sl]
    for r in range(1, 16):
      v = v + hsum_v[r, sl]
    acc_v[0, sl] = v * acc_v[0, sl] + acc_v[1, sl]
    return 0
  lax.fori_loop(0, RPT // 16, _fin, 0)

  @pl.when(cid == 0)
  def _():
    pltpu.sync_copy(acc_v.at[0], out.at[pl.ds(sid * RPT, RPT)])


_sc_out = pl.kernel(
    _sc_out_body,
    out_type=jax.ShapeDtypeStruct((N_PAD,), jnp.float32),
    mesh=plsc.VectorSubcoreMesh(core_axis_name="c", subcore_axis_name="s"),
    compiler_params=pltpu.CompilerParams(needs_layout_passes=False,
                                         use_tc_tiling_on_sc=False),
    scratch_types=[
        pltpu.VMEM((N_PAD,), jnp.float32),
        pltpu.VMEM((EPT,), jnp.int32),
        pltpu.VMEM((EPT,), jnp.int32),
        pltpu.VMEM((N_PAD,), jnp.float32),
        pltpu.VMEM((16, RPT), jnp.float32),
        pltpu.VMEM((2, RPT), jnp.float32),
        pltpu.VMEM_SHARED((16, N_PAD), jnp.float32),
    ],
)


@jax.jit
def kernel(x, edge_index, W1l, b1, W1r, W2l, b2, W2r):
  pad = E_PAD - edge_index.shape[1]
  pad_blk = jnp.concatenate(
      [jnp.zeros((1, pad), jnp.int32), jnp.full((1, pad), N, jnp.int32)],
      axis=0)
  ei = jnp.concatenate([edge_index, pad_blk], axis=1)
  ed3 = ei.reshape(2, 16, NCH, CH)
  ed2 = ei.reshape(2, 16, EPT)

  xbf = jnp.concatenate([x[:, :HH], x[:, HH:]], axis=0).astype(jnp.bfloat16)

  agg, deg = _sc_agg(xbf, ed3)

  s, u, invd = _tc_mid(agg, deg[:, :, None], x, W1l.T, W1r.T, b1[None, :],
                       W2l.T, W2r.T, b2[None, :])

  out = _sc_out(s.reshape(N_PAD), u.reshape(N_PAD), invd.reshape(N_PAD), ed2)
  return out[:N]


# speedup vs baseline: 1.1225x; 1.1225x over previous
import jax
import jax.numpy as jnp
from jax import lax
from jax.experimental import pallas as pl
from jax.experimental.pallas import tpu as pltpu
from jax.experimental.pallas import tpu_sc as plsc

N = 10000
F = 256
H = 256
HH = 128
N_PAD = 10240
E_PAD = 163840
CH = 128
NCH = 80
NB = 8
LEAD = 4
EPT = NCH * CH
RPT = N_PAD // 16


def _z16():
  return jnp.zeros((16,), jnp.float32)


def _sc_agg_body(xcat, src3, dst3, agg_out, deg_out,
                 ed_v, rows_v, misc_v, agg_sp, deg_sp, sem, sem_s):
  cid = lax.axis_index("c")
  sid = lax.axis_index("s")

  pltpu.sync_copy(src3.at[sid], ed_v.at[0])
  pltpu.sync_copy(dst3.at[sid], ed_v.at[1])

  def _zrow(j, _):
    def _zlane(l, _):
      rows_v[0, j, pl.ds(l * 32, 32)] = jnp.zeros((32,), jnp.bfloat16)
      return 0
    return lax.fori_loop(0, HH // 32, _zlane, 0)
  lax.fori_loop(0, CH, _zrow, 0)

  def _zv(k, _):
    misc_v[pl.ds(k * 16, 16)] = _z16()
    return 0
  lax.fori_loop(0, RPT // 16, _zv, 0)

  def _ov(k, _):
    misc_v[pl.ds(RPT + k * 16, 16)] = jnp.ones((16,), jnp.float32)
    return 0
  lax.fori_loop(0, CH // 16, _ov, 0)

  for r in range(RPT // CH):
    pltpu.sync_copy(rows_v.at[0], agg_sp.at[pl.ds(sid * RPT + r * CH, CH)])
  pltpu.sync_copy(misc_v.at[pl.ds(0, RPT)], deg_sp.at[pl.ds(sid * RPT, RPT)])

  off = (cid * N).astype(jnp.int32)
  def _adjj(j, _):
    def _adjl(l, _):
      ed_v[0, j, pl.ds(l * 16, 16)] = ed_v[0, j, pl.ds(l * 16, 16)] + off
      return 0
    return lax.fori_loop(0, CH // 16, _adjl, 0)
  lax.fori_loop(0, NCH, _adjj, 0)

  plsc.subcore_barrier()

  for b in range(LEAD):
    pltpu.async_copy(xcat.at[ed_v.at[0, b]], rows_v.at[b], sem)

  def _group(g, _):
    for b in range(NB):
      c = g * NB + b
      cl = c + LEAD
      bl = (b + LEAD) % NB

      @pl.when(cl < NCH)
      def _():
        @pl.when(c >= NB - LEAD)
        def _():
          pltpu.make_async_copy(rows_v.at[0], agg_sp.at[ed_v.at[1, 0]],
                                sem_s).wait()
        pltpu.async_copy(xcat.at[ed_v.at[0, cl]], rows_v.at[bl], sem)

      pltpu.make_async_copy(xcat.at[ed_v.at[0, c]], rows_v.at[b],
                            sem).wait()
      pltpu.async_copy(rows_v.at[b], agg_sp.at[ed_v.at[1, c]], sem_s,
                       add=True)
    return 0

  lax.fori_loop(0, NCH // NB, _group, 0)

  for _ in range(NB):
    pltpu.make_async_copy(rows_v.at[0], agg_sp.at[ed_v.at[1, 0]],
                          sem_s).wait()

  def _deg(j, _):
    pltpu.sync_copy(misc_v.at[pl.ds(RPT, CH)], deg_sp.at[ed_v.at[1, j]],
                    add=True)
    return 0
  lax.fori_loop(cid * (NCH // 2), (cid + 1) * (NCH // 2), _deg, 0)

  plsc.subcore_barrier()

  pltpu.sync_copy(agg_sp.at[pl.ds(sid * RPT, RPT)],
                  agg_out.at[cid, pl.ds(sid * RPT, RPT)])
  pltpu.sync_copy(deg_sp.at[pl.ds(sid * RPT, RPT)],
                  deg_out.at[cid, pl.ds(sid * RPT, RPT)])


_sc_agg = pl.kernel(
    _sc_agg_body,
    out_type=(
        jax.ShapeDtypeStruct((2, N_PAD, HH), jnp.bfloat16),
        jax.ShapeDtypeStruct((2, N_PAD), jnp.float32),
    ),
    mesh=plsc.VectorSubcoreMesh(core_axis_name="c", subcore_axis_name="s"),
    compiler_params=pltpu.CompilerParams(needs_layout_passes=False,
                                         use_tc_tiling_on_sc=False),
    scratch_types=[
        pltpu.VMEM((2, NCH, CH), jnp.int32),
        pltpu.VMEM((NB, CH, HH), jnp.bfloat16),
        pltpu.VMEM((RPT + CH,), jnp.float32),
        pltpu.VMEM_SHARED((N_PAD, HH), jnp.bfloat16),
        pltpu.VMEM_SHARED((N_PAD,), jnp.float32),
        pltpu.SemaphoreType.DMA,
        pltpu.SemaphoreType.DMA,
    ],
)


def _tc_mid_body(agg, deg, x, w1lt, w1rt, b1, w2lt, w2rt, b2,
                 s_out, u_out, invd_out):
  d = deg[0] + deg[1]
  invd = 1.0 / jnp.maximum(d, 1.0)
  m0 = agg[0].astype(jnp.float32) * invd
  m1 = agg[1].astype(jnp.float32) * invd
  a = jnp.dot(x[...], w1rt[...], preferred_element_type=jnp.float32)
  a = a + jnp.dot(m0, w1lt[0:HH, :], preferred_element_type=jnp.float32)
  a = a + jnp.dot(m1, w1lt[HH:F, :], preferred_element_type=jnp.float32)
  h = jnp.maximum(a + b1[...], 0.0)
  s_out[...] = jnp.dot(h, w2lt[...], preferred_element_type=jnp.float32)
  u_out[...] = jnp.dot(h, w2rt[...], preferred_element_type=jnp.float32) + b2[...]
  invd_out[...] = invd


_BM = 2000


def _tc_mid(agg, deg, x, w1lt, w1rt, b1, w2lt, w2rt, b2):
  grid = (N // _BM,)
  row = lambda i: (i, 0)
  rep = lambda i: (0, 0)
  return pl.pallas_call(
      _tc_mid_body,
      grid=grid,
      in_specs=[
          pl.BlockSpec((2, _BM, HH), lambda i: (0, i, 0)),
          pl.BlockSpec((2, _BM, 1), lambda i: (0, i, 0)),
          pl.BlockSpec((_BM, F), row),
          pl.BlockSpec((F, H), rep),
          pl.BlockSpec((F, H), rep),
          pl.BlockSpec((1, H), rep),
          pl.BlockSpec((H, 1), rep),
          pl.BlockSpec((H, 1), rep),
          pl.BlockSpec((1, 1), rep),
      ],
      out_specs=[
          pl.BlockSpec((_BM, 1), row),
          pl.BlockSpec((_BM, 1), row),
          pl.BlockSpec((_BM, 1), row),
      ],
      out_shape=[
          jax.ShapeDtypeStruct((N_PAD, 1), jnp.float32),
          jax.ShapeDtypeStruct((N_PAD, 1), jnp.float32),
          jax.ShapeDtypeStruct((N_PAD, 1), jnp.float32),
      ],
  )(agg, deg, x, w1lt, w1rt, b1, w2lt, w2rt, b2)


def _sc_out_body(s_pad, u_pad, invd_pad, src2, dst2, out,
                 s_v, src_v, dst_v, hist_v, hsum_v, acc_v, stage_sp):
  cid = lax.axis_index("c")
  sid = lax.axis_index("s")

  pltpu.sync_copy(s_pad, s_v)
  pltpu.sync_copy(src2.at[sid], src_v)
  pltpu.sync_copy(dst2.at[sid], dst_v)

  def _zhist(k, _):
    hist_v[pl.ds(k * 16, 16)] = _z16()
    return 0
  lax.fori_loop(0, N_PAD // 16, _zhist, 0)

  def _edge(k, _):
    idx = src_v[pl.ds(k * 16, 16)]
    d = dst_v[pl.ds(k * 16, 16)]
    v = plsc.load_gather(s_v, [idx])
    plsc.addupdate_scatter(hist_v, [d], v)
    return 0
  lax.fori_loop(0, EPT // 16, _edge, 0)

  pltpu.sync_copy(hist_v, stage_sp.at[sid])
  plsc.subcore_barrier()

  for r in range(16):
    pltpu.sync_copy(stage_sp.at[r, pl.ds(sid * RPT, RPT)], hsum_v.at[r])
  pltpu.sync_copy(invd_pad.at[pl.ds(sid * RPT, RPT)], acc_v.at[0])
  pltpu.sync_copy(u_pad.at[pl.ds(sid * RPT, RPT)], acc_v.at[1])

  def _fin(k, _):
    sl = pl.ds(k * 16, 16)
    v = hsum_v[0, sl]
    for r in range(1, 16):
      v = v + hsum_v[r, sl]
    acc_v[0, sl] = v * acc_v[0, sl] + acc_v[1, sl]
    return 0
  lax.fori_loop(0, RPT // 16, _fin, 0)

  @pl.when(cid == 0)
  def _():
    pltpu.sync_copy(acc_v.at[0], out.at[pl.ds(sid * RPT, RPT)])


_sc_out = pl.kernel(
    _sc_out_body,
    out_type=jax.ShapeDtypeStruct((N_PAD,), jnp.float32),
    mesh=plsc.VectorSubcoreMesh(core_axis_name="c", subcore_axis_name="s"),
    compiler_params=pltpu.CompilerParams(needs_layout_passes=False,
                                         use_tc_tiling_on_sc=False),
    scratch_types=[
        pltpu.VMEM((N_PAD,), jnp.float32),
        pltpu.VMEM((EPT,), jnp.int32),
        pltpu.VMEM((EPT,), jnp.int32),
        pltpu.VMEM((N_PAD,), jnp.float32),
        pltpu.VMEM((16, RPT), jnp.float32),
        pltpu.VMEM((2, RPT), jnp.float32),
        pltpu.VMEM_SHARED((16, N_PAD), jnp.float32),
    ],
)


@jax.jit
def kernel(x, edge_index, W1l, b1, W1r, W2l, b2, W2r):
  src = edge_index[0]
  dst = edge_index[1]
  pad = E_PAD - src.shape[0]
  src_p = jnp.concatenate([src, jnp.zeros((pad,), jnp.int32)])
  dst_p = jnp.concatenate([dst, jnp.full((pad,), N, jnp.int32)])
  src3 = src_p.reshape(16, NCH, CH)
  dst3 = dst_p.reshape(16, NCH, CH)
  src2 = src_p.reshape(16, EPT)
  dst2 = dst_p.reshape(16, EPT)

  xbf = jnp.concatenate([x[:, :HH], x[:, HH:]], axis=0).astype(jnp.bfloat16)

  agg, deg = _sc_agg(xbf, src3, dst3)

  s, u, invd = _tc_mid(agg, deg[:, :, None], x, W1l.T, W1r.T, b1[None, :],
                       W2l.T, W2r.T, b2[None, :])

  out = _sc_out(s.reshape(N_PAD), u.reshape(N_PAD), invd.reshape(N_PAD),
                src2, dst2)
  return out[:N]
